# Initial kernel scaffold; baseline (speedup 1.0000x reference)
import jax
import jax.numpy as jnp
from jax.experimental import pallas as pl


def kernel(args, x, rel_matrix, A1, edge_index, train_model, Kk_W1, Kk_W2, FN_W, FN_b, Wq_d, Wk_d, Wv_d, GT_Wq, GT_Wk, GT_Wv, GT_Wo, M1_W, M1_b, M2_W, M2_b, M3_W, M3_b, M4_W):
    raise NotImplementedError("write your pallas kernel here")


# speedup vs baseline: 1.0616x; 1.0616x over previous
import functools

import jax
import jax.numpy as jnp
import numpy as np
from jax.experimental import pallas as pl
from jax.experimental.pallas import tpu as pltpu

N = 10000
E = 160000
IN = 1546
GNN_HID = 1024
HID = 512
FOUT = 128
HEADS = 8
NR = 6000
ND = 4000
KP = 16384
SA = 2
DH = HID // HEADS


def _lrelu(x):
    return jnp.where(x >= 0, x, 0.01 * x)


def _mlp_body(feats_ref, w1_ref, b1_ref, w2_ref, b2_ref, w3_ref, b3_ref,
              w4_ref, lab16_ref, sel_ref, pred_ref, lab_ref):
    f = feats_ref[...]
    z = _lrelu(jnp.dot(f, w1_ref[...], preferred_element_type=jnp.float32)
               + b1_ref[...])
    z = _lrelu(jnp.dot(z, w2_ref[...], preferred_element_type=jnp.float32)
               + b2_ref[...])
    z = _lrelu(jnp.dot(z, w3_ref[...], preferred_element_type=jnp.float32)
               + b3_ref[...])
    pred_ref[...] = jax.nn.sigmoid(
        jnp.dot(z, w4_ref[...], preferred_element_type=jnp.float32))
    lab_ref[...] = jnp.sum(lab16_ref[...].astype(jnp.float32) * sel_ref[...],
                           axis=1, keepdims=True)


def _mlp_head(feats, M1_W, M1_b, M2_W, M2_b, M3_W, M3_b, M4_W, lab16, sel):
    w1 = jnp.pad(M1_W, ((0, 0), (0, 896 - 832)))
    b1 = jnp.pad(M1_b, (0, 896 - 832)).reshape(1, 896)
    w2 = jnp.pad(M2_W, ((0, 896 - 832), (0, 512 - 416)))
    b2 = jnp.pad(M2_b, (0, 512 - 416)).reshape(1, 512)
    w3 = jnp.pad(M3_W, ((0, 512 - 416), (0, 384 - 277)))
    b3 = jnp.pad(M3_b, (0, 384 - 277)).reshape(1, 384)
    w4 = jnp.pad(M4_W, ((0, 384 - 277), (0, 127)))

    BR = 1024
    grid = (KP // BR,)
    pred128, lab = pl.pallas_call(
        _mlp_body,
        grid=grid,
        in_specs=[
            pl.BlockSpec((BR, 1664), lambda i: (i, 0)),
            pl.BlockSpec((1664, 896), lambda i: (0, 0)),
            pl.BlockSpec((1, 896), lambda i: (0, 0)),
            pl.BlockSpec((896, 512), lambda i: (0, 0)),
            pl.BlockSpec((1, 512), lambda i: (0, 0)),
            pl.BlockSpec((512, 384), lambda i: (0, 0)),
            pl.BlockSpec((1, 384), lambda i: (0, 0)),
            pl.BlockSpec((384, 128), lambda i: (0, 0)),
            pl.BlockSpec((BR, 16), lambda i: (i, 0)),
            pl.BlockSpec((BR, 16), lambda i: (i, 0)),
        ],
        out_specs=[
            pl.BlockSpec((BR, 128), lambda i: (i, 0)),
            pl.BlockSpec((BR, 1), lambda i: (i, 0)),
        ],
        out_shape=[
            jax.ShapeDtypeStruct((KP, 128), jnp.float32),
            jax.ShapeDtypeStruct((KP, 1), jnp.float32),
        ],
    )(feats, w1, b1, w2, b2, w3, b3, w4, lab16, sel)
    return pred128[:, :1], lab[:, 0]


def kernel(args, x, rel_matrix, A1, edge_index, train_model, Kk_W1, Kk_W2,
           FN_W, FN_b, Wq_d, Wk_d, Wv_d, GT_Wq, GT_Wk, GT_Wv, GT_Wo,
           M1_W, M1_b, M2_W, M2_b, M3_W, M3_b, M4_W):
    src, dst = edge_index[0], edge_index[1]

    hW1 = x @ Kk_W1
    agg1 = jax.ops.segment_sum(hW1[src] * A1[:, None], dst, num_segments=N)
    h1 = _lrelu(hW1 + agg1)
    hW2 = h1 @ Kk_W2
    agg2 = jax.ops.segment_sum(hW2[src] * A1[:, None], dst, num_segments=N)
    out1 = _lrelu(hW2 + agg2)

    x1 = x @ FN_W + FN_b
    qh = (x1 @ Wq_d).reshape(N, HEADS, DH)
    kh = (x1 @ Wk_d).reshape(N, HEADS, DH)
    vh = (x1 @ Wv_d).reshape(N, HEADS, DH)
    offs = list(range(-3, 4))
    scores = jnp.stack([jnp.sum(qh * jnp.roll(kh, o, axis=0), axis=-1)
                        for o in offs], axis=0) / np.sqrt(DH)
    attn = jax.nn.softmax(scores, axis=0)
    out2 = sum(attn[i][:, :, None] * jnp.roll(vh, offs[i], axis=0)
               for i in range(len(offs)))
    out2 = out2.reshape(N, HID)

    l = SA - 1
    Q = (x @ GT_Wq[l]).reshape(N, HEADS, DH)
    K_ = (x @ GT_Wk[l]).reshape(N, HEADS, DH)
    V = (x @ GT_Wv[l]).reshape(N, HEADS, DH)
    e = jnp.sum(Q[dst] * K_[src], axis=-1) / np.sqrt(DH)
    ex = jnp.exp(e)
    den = jax.ops.segment_sum(ex, dst, num_segments=N)
    u = jax.ops.segment_sum(ex[:, :, None] * V[src], dst, num_segments=N)
    aggv = (u / (den + 1e-9)[:, :, None]).reshape(N, HID)
    ht = aggv @ GT_Wo[l]

    outputs = _lrelu(jnp.concatenate([out1, out2, ht], axis=1))

    idx = np.arange(KP)
    rows = idx % NR
    cols = (idx * 7) % ND
    feats = outputs[rows] * outputs[NR + cols]

    flat = rows.astype(np.int64) * ND + cols
    r16 = (flat // 16).astype(np.int32)
    c16 = (flat % 16).astype(np.int32)
    rel16 = rel_matrix.reshape(-1, 16)
    lab16 = rel16[r16]
    sel = jnp.asarray(np.eye(16, dtype=np.float32)[c16])

    pred, labels = _mlp_head(feats, M1_W, M1_b, M2_W, M2_b, M3_W, M3_b,
                             M4_W, lab16, sel)
    return pred, labels


# speedup vs baseline: 1.1587x; 1.0915x over previous
import functools

import jax
import jax.numpy as jnp
import numpy as np
from jax import lax
from jax.experimental import pallas as pl
from jax.experimental.pallas import tpu as pltpu
from jax.experimental.pallas import tpu_sc as plsc

N = 10000
NP = 10240
E = 160000
IN = 1546
INP = 1664
GNN_HID = 1024
HID = 512
FOUT = 128
HEADS = 8
NR = 6000
ND = 4000
KP = 16384
SA = 2
DH = HID // HEADS

NTILES = 32
NSUB = 16
ROWS_PER_TILE = NP // NSUB


def _lrelu(x):
    return jnp.where(x >= 0, x, 0.01 * x)


_SEG_B = 128
EP = 163840
_SEG_NB = EP // (_SEG_B * NTILES)


def _seg_body(C, mode, tbl, srcv, dstv, wv, zeros_hbm, out0, out1,
              idx_s, idx_d, wbuf, rows, zbuf, acc, sem):
    ci_core = lax.axis_index("c")
    sid = lax.axis_index("s")
    wid = ci_core * NSUB + sid
    pltpu.sync_copy(zeros_hbm, zbuf)
    row0 = sid * ROWS_PER_TILE

    for c in range(C):
        for r in range(5):
            pltpu.sync_copy(zbuf, acc.at[pl.ds(row0 + r * 128, 128)])
        plsc.subcore_barrier()

        def batch_body(t, _, c=c):
            gbase = (wid * _SEG_NB + t) * _SEG_B
            pltpu.sync_copy(srcv.at[pl.ds(gbase, _SEG_B)], idx_s)
            pltpu.sync_copy(dstv.at[pl.ds(gbase, _SEG_B)], idx_d)
            if mode == "scalar":
                pltpu.sync_copy(wv.at[pl.ds(gbase, _SEG_B)], wbuf)
            else:
                pltpu.sync_copy(wv.at[pl.ds(gbase, _SEG_B), :], wbuf)
            pltpu.async_copy(tbl.at[c].at[idx_s], rows, sem).wait()

            def group_body(g, _):
                e0 = g * 16
                if mode == "scalar":
                    aw = wbuf[pl.ds(e0, 16)]
                for i in range(16):
                    e = e0 + i
                    if mode == "scalar":
                        a0 = aw[i]
                        a1 = a0
                    else:
                        wrow = wbuf[e]
                        a0 = wrow[2 * c]
                        a1 = wrow[2 * c + 1]
                    for j in range(8):
                        sl = pl.ds(j * 16, 16)
                        a = a0 if j < 4 else a1
                        rows[e, sl] = rows[e, sl] * a
                return 0

            lax.fori_loop(0, _SEG_B // 16, group_body, 0)
            pltpu.sync_copy(rows, acc.at[idx_d], add=True)
            return 0

        lax.fori_loop(0, _SEG_NB, batch_body, 0)
        plsc.subcore_barrier()

        src_slice = acc.at[pl.ds(row0, ROWS_PER_TILE)]

        @pl.when(ci_core == 0)
        def _():
            pltpu.sync_copy(src_slice, out0.at[c].at[pl.ds(row0, ROWS_PER_TILE)])

        @pl.when(ci_core == 1)
        def _():
            pltpu.sync_copy(src_slice, out1.at[c].at[pl.ds(row0, ROWS_PER_TILE)])


def _seg_sc(tbl, srcv, dstv, wv, mode):
    C = tbl.shape[0]
    mesh = plsc.VectorSubcoreMesh(core_axis_name="c", subcore_axis_name="s")
    wshape = (_SEG_B,) if mode == "scalar" else (_SEG_B, 16)
    zeros = jnp.zeros((128, 128), jnp.float32)
    body = functools.partial(_seg_body, C, mode)
    f = pl.kernel(
        body,
        out_type=[jax.ShapeDtypeStruct((C, NP, 128), jnp.float32)] * 2,
        mesh=mesh,
        scratch_types=[
            pltpu.VMEM((_SEG_B,), jnp.int32),
            pltpu.VMEM((_SEG_B,), jnp.int32),
            pltpu.VMEM(wshape, jnp.float32),
            pltpu.VMEM((_SEG_B, 128), jnp.float32),
            pltpu.VMEM((128, 128), jnp.float32),
            pltpu.VMEM_SHARED((NP, 128), jnp.float32),
            pltpu.SemaphoreType.DMA,
        ],
    )
    return f(tbl, srcv, dstv, wv, zeros)


def _mm_chunk_body(x_ref, w_ref, o_ref):
    j = pl.program_id(1)
    wsl = w_ref[:, pl.ds(j * 128, 128)]
    o_ref[0] = jnp.dot(x_ref[...], wsl, preferred_element_type=jnp.float32)


def _mm_chunk(x, w, BM):
    M, K = x.shape
    CO = w.shape[1] // 128
    grid = (M // BM, CO)
    return pl.pallas_call(
        _mm_chunk_body,
        grid=grid,
        in_specs=[
            pl.BlockSpec((BM, K), lambda i, j: (i, 0)),
            pl.BlockSpec((K, w.shape[1]), lambda i, j: (0, 0)),
        ],
        out_specs=pl.BlockSpec((1, BM, 128), lambda i, j: (j, i, 0)),
        out_shape=jax.ShapeDtypeStruct((CO, M, 128), jnp.float32),
    )(x, w)


def _act1_body(xw_ref, p0_ref, p1_ref, o_ref):
    o_ref[...] = _lrelu(xw_ref[...] + p0_ref[...] + p1_ref[...])


def _act1(xw, p0, p1):
    C = p0.shape[0]
    BM = 1000
    grid = (C, N // BM)
    blk = pl.BlockSpec((1, BM, 128), lambda c, i: (c, i, 0))
    return pl.pallas_call(
        _act1_body,
        grid=grid,
        in_specs=[blk, blk, blk],
        out_specs=blk,
        out_shape=jax.ShapeDtypeStruct((C, N, 128), jnp.float32),
    )(xw, p0, p1)


def _mm_l2_body(h_ref, w_ref, o_ref):
    j = pl.program_id(1)
    jds = pl.ds(j * 128, 128)
    acc = jnp.zeros((h_ref.shape[1], 128), jnp.float32)
    for k in range(8):
        acc += jnp.dot(h_ref[k], w_ref[k * 128:(k + 1) * 128, jds],
                       preferred_element_type=jnp.float32)
    o_ref[0] = acc


def _mm_l2(h1t, W2):
    BM = 1000
    grid = (N // BM, 8)
    return pl.pallas_call(
        _mm_l2_body,
        grid=grid,
        in_specs=[
            pl.BlockSpec((8, BM, 128), lambda i, j: (0, i, 0)),
            pl.BlockSpec((1024, 1024), lambda i, j: (0, 0)),
        ],
        out_specs=pl.BlockSpec((1, BM, 128), lambda i, j: (j, i, 0)),
        out_shape=jax.ShapeDtypeStruct((8, N, 128), jnp.float32),
    )(h1t, W2)


def _act2_body(hw_ref, p0_ref, p1_ref, o_ref):
    o_ref[...] = _lrelu(hw_ref[0] + p0_ref[0] + p1_ref[0])


def _act2(hw, p0, p1):
    BM = 1000
    grid = (8, N // BM)
    blk = pl.BlockSpec((1, BM, 128), lambda c, i: (c, i, 0))
    return pl.pallas_call(
        _act2_body,
        grid=grid,
        in_specs=[blk, blk, blk],
        out_specs=pl.BlockSpec((BM, 128), lambda c, i: (i, c)),
        out_shape=jax.ShapeDtypeStruct((N, 1024), jnp.float32),
    )(hw, p0, p1)


def _mlp_body(feats_ref, w1_ref, b1_ref, w2_ref, b2_ref, w3_ref, b3_ref,
              w4_ref, lab16_ref, sel_ref, pred_ref, lab_ref):
    f = feats_ref[...]
    z = _lrelu(jnp.dot(f, w1_ref[...], preferred_element_type=jnp.float32)
               + b1_ref[...])
    z = _lrelu(jnp.dot(z, w2_ref[...], preferred_element_type=jnp.float32)
               + b2_ref[...])
    z = _lrelu(jnp.dot(z, w3_ref[...], preferred_element_type=jnp.float32)
               + b3_ref[...])
    pred_ref[...] = jax.nn.sigmoid(
        jnp.dot(z, w4_ref[...], preferred_element_type=jnp.float32))
    lab_ref[...] = jnp.sum(lab16_ref[...].astype(jnp.float32) * sel_ref[...],
                           axis=1, keepdims=True)


def _mlp_head(feats, M1_W, M1_b, M2_W, M2_b, M3_W, M3_b, M4_W, lab16, sel):
    w1 = jnp.pad(M1_W, ((0, 0), (0, 896 - 832)))
    b1 = jnp.pad(M1_b, (0, 896 - 832)).reshape(1, 896)
    w2 = jnp.pad(M2_W, ((0, 896 - 832), (0, 512 - 416)))
    b2 = jnp.pad(M2_b, (0, 512 - 416)).reshape(1, 512)
    w3 = jnp.pad(M3_W, ((0, 512 - 416), (0, 384 - 277)))
    b3 = jnp.pad(M3_b, (0, 384 - 277)).reshape(1, 384)
    w4 = jnp.pad(M4_W, ((0, 384 - 277), (0, 127)))

    BR = 1024
    grid = (KP // BR,)
    pred128, lab = pl.pallas_call(
        _mlp_body,
        grid=grid,
        in_specs=[
            pl.BlockSpec((BR, 1664), lambda i: (i, 0)),
            pl.BlockSpec((1664, 896), lambda i: (0, 0)),
            pl.BlockSpec((1, 896), lambda i: (0, 0)),
            pl.BlockSpec((896, 512), lambda i: (0, 0)),
            pl.BlockSpec((1, 512), lambda i: (0, 0)),
            pl.BlockSpec((512, 384), lambda i: (0, 0)),
            pl.BlockSpec((1, 384), lambda i: (0, 0)),
            pl.BlockSpec((384, 128), lambda i: (0, 0)),
            pl.BlockSpec((BR, 16), lambda i: (i, 0)),
            pl.BlockSpec((BR, 16), lambda i: (i, 0)),
        ],
        out_specs=[
            pl.BlockSpec((BR, 128), lambda i: (i, 0)),
            pl.BlockSpec((BR, 1), lambda i: (i, 0)),
        ],
        out_shape=[
            jax.ShapeDtypeStruct((KP, 128), jnp.float32),
            jax.ShapeDtypeStruct((KP, 1), jnp.float32),
        ],
    )(feats, w1, b1, w2, b2, w3, b3, w4, lab16, sel)
    return pred128[:, :1], lab[:, 0]


def kernel(args, x, rel_matrix, A1, edge_index, train_model, Kk_W1, Kk_W2,
           FN_W, FN_b, Wq_d, Wk_d, Wv_d, GT_Wq, GT_Wk, GT_Wv, GT_Wo,
           M1_W, M1_b, M2_W, M2_b, M3_W, M3_b, M4_W):
    src, dst = edge_index[0], edge_index[1]

    srcp = jnp.pad(src, (0, EP - E))
    dstp = jnp.pad(dst, (0, EP - E))
    A1p = jnp.pad(A1, (0, EP - E))

    x_pad = jnp.pad(x, ((0, NP - N), (0, INP - IN)))
    W1p = jnp.pad(Kk_W1, ((0, INP - IN), (0, 0)))
    XW = _mm_chunk(x_pad, W1p, BM=2048)
    XWn = XW[:, :N]
    p0, p1 = _seg_sc(XWn, srcp, dstp, A1p, "scalar")
    h1t = _act1(XWn, p0, p1)
    hW2t = _mm_l2(h1t, Kk_W2)
    q0, q1 = _seg_sc(hW2t, srcp, dstp, A1p, "scalar")
    out1 = _act2(hW2t, q0, q1)

    x1 = x @ FN_W + FN_b
    qh = (x1 @ Wq_d).reshape(N, HEADS, DH)
    kh = (x1 @ Wk_d).reshape(N, HEADS, DH)
    vh = (x1 @ Wv_d).reshape(N, HEADS, DH)
    offs = list(range(-3, 4))
    scores = jnp.stack([jnp.sum(qh * jnp.roll(kh, o, axis=0), axis=-1)
                        for o in offs], axis=0) / np.sqrt(DH)
    attn = jax.nn.softmax(scores, axis=0)
    out2 = sum(attn[i][:, :, None] * jnp.roll(vh, offs[i], axis=0)
               for i in range(len(offs)))
    out2 = out2.reshape(N, HID)

    l = SA - 1
    Q = (x @ GT_Wq[l]).reshape(N, HEADS, DH)
    K_ = (x @ GT_Wk[l]).reshape(N, HEADS, DH)
    V = (x @ GT_Wv[l]).reshape(N, HEADS, DH)
    e = jnp.sum(Q[dst] * K_[src], axis=-1) / np.sqrt(DH)
    ex = jnp.exp(e)
    den = jax.ops.segment_sum(ex, dst, num_segments=N)
    u = jax.ops.segment_sum(ex[:, :, None] * V[src], dst, num_segments=N)
    aggv = (u / (den + 1e-9)[:, :, None]).reshape(N, HID)
    ht = aggv @ GT_Wo[l]

    outputs = _lrelu(jnp.concatenate([out1, out2, ht], axis=1))

    idx = np.arange(KP)
    rows = idx % NR
    cols = (idx * 7) % ND
    feats = outputs[rows] * outputs[NR + cols]

    flat = rows.astype(np.int64) * ND + cols
    r16 = (flat // 16).astype(np.int32)
    c16 = (flat % 16).astype(np.int32)
    rel16 = rel_matrix.reshape(-1, 16)
    lab16 = rel16[r16]
    sel = jnp.asarray(np.eye(16, dtype=np.float32)[c16])

    pred, labels = _mlp_head(feats, M1_W, M1_b, M2_W, M2_b, M3_W, M3_b,
                             M4_W, lab16, sel)
    return pred, labels
